# baseline (speedup 1.0000x reference)
import math
from functools import partial

import jax
import jax.numpy as jnp
from jax import lax
from jax.experimental import pallas as pl
from jax.experimental.pallas import tpu as pltpu

BN_EPS = 1e-5
VMEM_LIMIT = 64 * 1024 * 1024


def _bn_ss(st_ref, g_ref, b_ref, cnt):
    s1 = jnp.sum(st_ref[:, 0, :], axis=0, keepdims=True)
    s2 = jnp.sum(st_ref[:, 1, :], axis=0, keepdims=True)
    mean = s1 / cnt
    var = jnp.maximum(s2 / cnt - mean * mean, 0.0)
    scale = g_ref[...] * lax.rsqrt(var + BN_EPS)
    shift = b_ref[...] - mean * scale
    return scale, shift


def _conv9_stacked(xpad_ref, w_ref, ML, W2):
    Cout = w_ref.shape[-1]
    acc = jnp.zeros((ML, Cout), jnp.float32)
    for t in range(9):
        off = (t // 3) * W2 + (t % 3)
        acc = acc + jnp.dot(xpad_ref[pl.ds(off, ML), :], w_ref[t],
                            preferred_element_type=jnp.float32)
    return acc


def _store_stats(acc, B, ML, W, W2, Mp, M, y_ref, st_ref):
    y_ref[pl.ds(0, ML), :] = acc.astype(y_ref.dtype)
    r = lax.broadcasted_iota(jnp.int32, (M, 1), 0)
    cmask = (r % W2) < W
    for b in range(B):
        ymb = jnp.where(cmask, acc[b * Mp:b * Mp + M], 0.0)
        st_ref[b, 0:1, :] = jnp.sum(ymb, axis=0, keepdims=True)
        st_ref[b, 1:2, :] = jnp.sum(ymb * ymb, axis=0, keepdims=True)


def _conv1_kernel(x_ref, w_ref, y_ref, st_ref, xcol_ref, *, B, H, W):
    W2 = W + 2
    Mp = (H + 3) * W2
    M = H * W2
    ML = B * Mp - 3 * W2
    for dh in range(3):
        xcol_ref[:, 9 * dh:9 * (dh + 1)] = x_ref[pl.ds(dh * W2, ML), :]
    acc = jnp.dot(xcol_ref[...], w_ref[...], preferred_element_type=jnp.float32)
    _store_stats(acc, B, ML, W, W2, Mp, M, y_ref, st_ref)


def _pool_conv_kernel(prev_ref, st_ref_in, g_ref, b_ref, w_ref, y_ref, st_ref,
                      act_ref, rmax_ref, xpad_ref, *, B, Hp, Wp, Hc, Wc, cnt):
    scale, shift = _bn_ss(st_ref_in, g_ref, b_ref, cnt)
    W2c = Wc + 2
    Mpc = (Hc + 3) * W2c
    xpad_ref[...] = jnp.zeros_like(xpad_ref)

    for b in range(B):
        act_ref[...] = jnp.maximum(prev_ref[b, 0:Hp].astype(jnp.float32) * scale + shift, 0.0)
        cm = act_ref[:, pl.ds(0, Wc, stride=2), :]
        cm = jnp.maximum(cm, act_ref[:, pl.ds(1, Wc, stride=2), :])
        cm = jnp.maximum(cm, act_ref[:, pl.ds(2, Wc, stride=2), :])
        rmax_ref[...] = cm
        pooled = rmax_ref[pl.ds(0, Hc, stride=2), :, :]
        pooled = jnp.maximum(pooled, rmax_ref[pl.ds(1, Hc, stride=2), :, :])
        pooled = jnp.maximum(pooled, rmax_ref[pl.ds(2, Hc, stride=2), :, :])
        pooled = pooled.astype(jnp.bfloat16)
        base = b * Mpc
        for h in range(Hc):
            xpad_ref[pl.ds(base + (h + 1) * W2c + 1, Wc), :] = pooled[h]

    ML = B * Mpc - 3 * W2c
    acc = _conv9_stacked(xpad_ref, w_ref, ML, W2c)
    _store_stats(acc, B, ML, Wc, W2c, Mpc, Hc * W2c, y_ref, st_ref)


def _conv_kernel(prev_ref, st_ref_in, g_ref, b_ref, w_ref, y_ref, st_ref,
                 xpad_ref, *, B, H, W, cnt):
    W2 = W + 2
    M = H * W2
    Mp = (H + 3) * W2
    scale, shift = _bn_ss(st_ref_in, g_ref, b_ref, cnt)
    ML0 = B * Mp
    r = lax.broadcasted_iota(jnp.int32, (ML0, 1), 0)
    norm = jnp.where((r % W2) < W,
                     jnp.maximum(prev_ref[...].astype(jnp.float32) * scale + shift, 0.0),
                     0.0).astype(jnp.bfloat16)

    zb = jnp.zeros((W2 + 1, norm.shape[-1]), jnp.bfloat16)
    zt = jnp.zeros((2 * W2 - 1, norm.shape[-1]), jnp.bfloat16)
    for b in range(B):
        xpad_ref[pl.ds(b * Mp, W2 + 1), :] = zb
        xpad_ref[pl.ds(b * Mp + W2 + 1 + M, 2 * W2 - 1), :] = zt
        xpad_ref[pl.ds(b * Mp + W2 + 1, M), :] = norm[b * Mp:b * Mp + M]

    ML = B * Mp - 3 * W2
    acc = _conv9_stacked(xpad_ref, w_ref, ML, W2)
    _store_stats(acc, B, ML, W, W2, Mp, M, y_ref, st_ref)


def _head_kernel(prev_ref, st_ref_in, g_ref, b_ref, p_ref, out_ref,
                 *, B, M, Mp, cnt):
    scale, shift = _bn_ss(st_ref_in, g_ref, b_ref, cnt)
    for b in range(B):
        act = jnp.maximum(prev_ref[pl.ds(b * Mp, M), :].astype(jnp.float32) * scale + shift, 0.0)
        out_ref[b] = jnp.dot(p_ref[...], act, preferred_element_type=jnp.float32)


def _conv_first(xfold, w27, B, H, W, od):
    S = xfold.shape[0]
    Cout = w27.shape[-1]
    W2 = W + 2
    Mp = (H + 3) * W2
    ML = B * Mp - 3 * W2
    return pl.pallas_call(
        partial(_conv1_kernel, B=B, H=H, W=W),
        out_shape=(jax.ShapeDtypeStruct((S, B * Mp, Cout), od),
                   jax.ShapeDtypeStruct((S, B, 2, Cout), jnp.float32)),
        grid=(S,),
        in_specs=[pl.BlockSpec((None, B * Mp, 9), lambda n: (n, 0, 0)),
                  pl.BlockSpec((27, Cout), lambda n: (0, 0))],
        out_specs=(pl.BlockSpec((None, B * Mp, Cout), lambda n: (n, 0, 0)),
                   pl.BlockSpec((None, B, 2, Cout), lambda n: (n, 0, 0, 0))),
        scratch_shapes=[pltpu.VMEM((ML, 27), jnp.bfloat16)],
        compiler_params=pltpu.CompilerParams(
            dimension_semantics=("parallel",), vmem_limit_bytes=VMEM_LIMIT),
    )(xfold, w27)


def _fused_pool_conv(y_prev, st_prev, gamma, beta, w9, B, Hp, Wp, cnt, od):
    Cp = y_prev.shape[-1]
    Cout = w9.shape[-1]
    W2p = Wp + 2
    Hc, Wc = (Hp - 3) // 2 + 1, (Wp - 3) // 2 + 1
    W2c = Wc + 2
    Mpc = (Hc + 3) * W2c
    N_TOT = (y_prev.shape[0] * y_prev.shape[1]) // ((Hp + 3) * W2p)
    S = N_TOT // B
    prev5 = y_prev.reshape(S, B, Hp + 3, W2p, Cp)
    st_in = st_prev.reshape(N_TOT, 2, Cp)
    y, st = pl.pallas_call(
        partial(_pool_conv_kernel, B=B, Hp=Hp, Wp=Wp, Hc=Hc, Wc=Wc, cnt=cnt),
        out_shape=(jax.ShapeDtypeStruct((S, B * Mpc, Cout), od),
                   jax.ShapeDtypeStruct((S, B, 2, Cout), jnp.float32)),
        grid=(S,),
        in_specs=[pl.BlockSpec((None, B, Hp + 3, W2p, Cp),
                               lambda n: (n, 0, 0, 0, 0)),
                  pl.BlockSpec((N_TOT, 2, Cp), lambda n: (0, 0, 0)),
                  pl.BlockSpec((1, Cp), lambda n: (0, 0)),
                  pl.BlockSpec((1, Cp), lambda n: (0, 0)),
                  pl.BlockSpec((9, Cp, Cout), lambda n: (0, 0, 0))],
        out_specs=(pl.BlockSpec((None, B * Mpc, Cout), lambda n: (n, 0, 0)),
                   pl.BlockSpec((None, B, 2, Cout), lambda n: (n, 0, 0, 0))),
        scratch_shapes=[pltpu.VMEM((Hp, W2p, Cp), jnp.float32),
                        pltpu.VMEM((Hp, Wc, Cp), jnp.float32),
                        pltpu.VMEM((B * Mpc, Cp), jnp.bfloat16)],
        compiler_params=pltpu.CompilerParams(
            dimension_semantics=("parallel",), vmem_limit_bytes=VMEM_LIMIT),
    )(prev5, st_in, gamma, beta, w9)
    return y, st, Hc, Wc


def _fused_conv(y_prev, st_prev, gamma, beta, w9, B, H, W, cnt, od):
    Cp = y_prev.shape[-1]
    Cout = w9.shape[-1]
    W2 = W + 2
    Mp = (H + 3) * W2
    N_TOT = (y_prev.shape[0] * y_prev.shape[1]) // Mp
    S = N_TOT // B
    yv = y_prev.reshape(S, B * Mp, Cp)
    st_in = st_prev.reshape(N_TOT, 2, Cp)
    return pl.pallas_call(
        partial(_conv_kernel, B=B, H=H, W=W, cnt=cnt),
        out_shape=(jax.ShapeDtypeStruct((S, B * Mp, Cout), od),
                   jax.ShapeDtypeStruct((S, B, 2, Cout), jnp.float32)),
        grid=(S,),
        in_specs=[pl.BlockSpec((None, B * Mp, Cp), lambda n: (n, 0, 0)),
                  pl.BlockSpec((N_TOT, 2, Cp), lambda n: (0, 0, 0)),
                  pl.BlockSpec((1, Cp), lambda n: (0, 0)),
                  pl.BlockSpec((1, Cp), lambda n: (0, 0)),
                  pl.BlockSpec((9, Cp, Cout), lambda n: (0, 0, 0))],
        out_specs=(pl.BlockSpec((None, B * Mp, Cout), lambda n: (n, 0, 0)),
                   pl.BlockSpec((None, B, 2, Cout), lambda n: (n, 0, 0, 0))),
        scratch_shapes=[pltpu.VMEM((B * Mp, Cp), jnp.bfloat16)],
        compiler_params=pltpu.CompilerParams(
            dimension_semantics=("parallel",), vmem_limit_bytes=VMEM_LIMIT),
    )(yv, st_in, gamma, beta, w9)


def _head_pool(y_prev, st_prev, gamma, beta, pmat, B, H, W, cnt):
    C = y_prev.shape[-1]
    P, M = pmat.shape
    Mp = (H + 3) * (W + 2)
    N_TOT = (y_prev.shape[0] * y_prev.shape[1]) // Mp
    S = N_TOT // B
    yv = y_prev.reshape(S, B * Mp, C)
    st_in = st_prev.reshape(N_TOT, 2, C)
    return pl.pallas_call(
        partial(_head_kernel, B=B, M=M, Mp=Mp, cnt=cnt),
        out_shape=jax.ShapeDtypeStruct((S, B, P, C), jnp.float32),
        grid=(S,),
        in_specs=[pl.BlockSpec((None, B * Mp, C), lambda n: (n, 0, 0)),
                  pl.BlockSpec((N_TOT, 2, C), lambda n: (0, 0, 0)),
                  pl.BlockSpec((1, C), lambda n: (0, 0)),
                  pl.BlockSpec((1, C), lambda n: (0, 0)),
                  pl.BlockSpec((P, M), lambda n: (0, 0))],
        out_specs=pl.BlockSpec((None, B, P, C), lambda n: (n, 0, 0, 0)),
        compiler_params=pltpu.CompilerParams(
            dimension_semantics=("parallel",), vmem_limit_bytes=VMEM_LIMIT),
    )(yv, st_in, gamma, beta, pmat)


def kernel(conv_w_0, conv_w_1, conv_w_2, conv_w_3, conv_w_4, conv_w_5,
           gamma_0, gamma_1, gamma_2, gamma_3, gamma_4, gamma_5,
           beta_0, beta_1, beta_2, beta_3, beta_4, beta_5,
           fc_w_perm, fc_b, pool_mat, x):
    conv_w = [conv_w_0, conv_w_1, conv_w_2, conv_w_3, conv_w_4, conv_w_5]
    gammas = [gamma_0, gamma_1, gamma_2, gamma_3, gamma_4, gamma_5]
    betas = [beta_0, beta_1, beta_2, beta_3, beta_4, beta_5]

    N, Cin, H, W = x.shape
    B1, B2, B3, B46, BH = 1, 1, 8, 16, 16
    S1 = N // B1
    W2 = W + 2
    Mp = (H + 3) * W2

    xh = jnp.transpose(x, (0, 2, 3, 1)).astype(jnp.float32)
    xp = jnp.pad(xh, ((0, 0), (1, 2), (1, 1), (0, 0))).astype(jnp.bfloat16)
    xp = xp.reshape(N, Mp, Cin)
    xpb = jnp.pad(xp, ((0, 0), (0, 2), (0, 0)))
    xfold = jnp.concatenate(
        [xpb[:, 0:Mp], xpb[:, 1:Mp + 1], xpb[:, 2:Mp + 2]], axis=2)
    xfold = xfold.reshape(S1, B1 * Mp, 3 * Cin)
    w27 = conv_w[0].reshape(9 * Cin, conv_w[0].shape[-1])

    ODS = [jnp.float32, jnp.float32, jnp.float32,
           jnp.float32, jnp.float32, jnp.float32]
    y, st = _conv_first(xfold, w27, B1, H, W, ODS[0])
    h, w = H, W

    for i, bi in ((1, B2), (2, B3)):
        y, st, h, w = _fused_pool_conv(y, st, gammas[i - 1], betas[i - 1],
                                       conv_w[i], bi, h, w, float(N * h * w), ODS[i])

    for i in (3, 4, 5):
        y, st = _fused_conv(y, st, gammas[i - 1], betas[i - 1],
                            conv_w[i], B46, h, w, float(N * h * w), ODS[i])

    pooled = _head_pool(y, st, gammas[5], betas[5], pool_mat, BH, h, w,
                        float(N * h * w))

    flat = pooled.reshape(N, -1)
    return flat @ fc_w_perm + fc_b[None, :]


# speedup vs baseline: 1.0113x; 1.0113x over previous
import math
from functools import partial

import jax
import jax.numpy as jnp
from jax import lax
from jax.experimental import pallas as pl
from jax.experimental.pallas import tpu as pltpu

BN_EPS = 1e-5
VMEM_LIMIT = 64 * 1024 * 1024


def _bn_ss(st_ref, g_ref, b_ref, cnt):
    s1 = jnp.sum(st_ref[:, 0, :], axis=0, keepdims=True)
    s2 = jnp.sum(st_ref[:, 1, :], axis=0, keepdims=True)
    mean = s1 / cnt
    var = jnp.maximum(s2 / cnt - mean * mean, 0.0)
    scale = g_ref[...] * lax.rsqrt(var + BN_EPS)
    shift = b_ref[...] - mean * scale
    return scale, shift


def _conv9_stacked(xpad_ref, w_ref, ML, W2):
    Cout = w_ref.shape[-1]
    acc = jnp.zeros((ML, Cout), jnp.float32)
    for t in range(9):
        off = (t // 3) * W2 + (t % 3)
        acc = acc + jnp.dot(xpad_ref[pl.ds(off, ML), :], w_ref[t],
                            preferred_element_type=jnp.float32)
    return acc


def _store_stats(acc, B, ML, W, W2, Mp, M, y_ref, st_ref):
    y_ref[pl.ds(0, ML), :] = acc.astype(y_ref.dtype)
    r = lax.broadcasted_iota(jnp.int32, (M, 1), 0)
    cmask = (r % W2) < W
    for b in range(B):
        ymb = jnp.where(cmask, acc[b * Mp:b * Mp + M], 0.0)
        st_ref[b, 0:1, :] = jnp.sum(ymb, axis=0, keepdims=True)
        st_ref[b, 1:2, :] = jnp.sum(ymb * ymb, axis=0, keepdims=True)


def _conv1_kernel(x_ref, w_ref, y_ref, st_ref, xcol_ref, *, B, H, W):
    W2 = W + 2
    Mp = (H + 3) * W2
    M = H * W2
    ML = B * Mp - 3 * W2
    for dh in range(3):
        xcol_ref[:, 9 * dh:9 * (dh + 1)] = x_ref[pl.ds(dh * W2, ML), :]
    acc = jnp.dot(xcol_ref[...], w_ref[...], preferred_element_type=jnp.float32)
    _store_stats(acc, B, ML, W, W2, Mp, M, y_ref, st_ref)


def _pool_conv_kernel(prev_ref, st_ref_in, g_ref, b_ref, w_ref, y_ref, st_ref,
                      act_ref, rmax_ref, xpad_ref, *, B, Hp, Wp, Hc, Wc, cnt):
    scale, shift = _bn_ss(st_ref_in, g_ref, b_ref, cnt)
    W2c = Wc + 2
    Mpc = (Hc + 3) * W2c
    xpad_ref[...] = jnp.zeros_like(xpad_ref)

    for b in range(B):
        act_ref[...] = jnp.maximum(prev_ref[b, 0:Hp].astype(jnp.float32) * scale + shift, 0.0)
        cm = act_ref[:, pl.ds(0, Wc, stride=2), :]
        cm = jnp.maximum(cm, act_ref[:, pl.ds(1, Wc, stride=2), :])
        cm = jnp.maximum(cm, act_ref[:, pl.ds(2, Wc, stride=2), :])
        rmax_ref[...] = cm
        pooled = rmax_ref[pl.ds(0, Hc, stride=2), :, :]
        pooled = jnp.maximum(pooled, rmax_ref[pl.ds(1, Hc, stride=2), :, :])
        pooled = jnp.maximum(pooled, rmax_ref[pl.ds(2, Hc, stride=2), :, :])
        pooled = pooled.astype(jnp.bfloat16)
        base = b * Mpc
        for h in range(Hc):
            xpad_ref[pl.ds(base + (h + 1) * W2c + 1, Wc), :] = pooled[h]

    ML = B * Mpc - 3 * W2c
    acc = _conv9_stacked(xpad_ref, w_ref, ML, W2c)
    _store_stats(acc, B, ML, Wc, W2c, Mpc, Hc * W2c, y_ref, st_ref)


def _conv_kernel(prev_ref, st_ref_in, g_ref, b_ref, w_ref, y_ref, st_ref,
                 xpad_ref, *, B, H, W, cnt):
    W2 = W + 2
    M = H * W2
    Mp = (H + 3) * W2
    scale, shift = _bn_ss(st_ref_in, g_ref, b_ref, cnt)
    ML0 = B * Mp
    r = lax.broadcasted_iota(jnp.int32, (ML0, 1), 0)
    norm = jnp.where((r % W2) < W,
                     jnp.maximum(prev_ref[...].astype(jnp.float32) * scale + shift, 0.0),
                     0.0).astype(jnp.bfloat16)

    zb = jnp.zeros((W2 + 1, norm.shape[-1]), jnp.bfloat16)
    zt = jnp.zeros((2 * W2 - 1, norm.shape[-1]), jnp.bfloat16)
    for b in range(B):
        xpad_ref[pl.ds(b * Mp, W2 + 1), :] = zb
        xpad_ref[pl.ds(b * Mp + W2 + 1 + M, 2 * W2 - 1), :] = zt
        xpad_ref[pl.ds(b * Mp + W2 + 1, M), :] = norm[b * Mp:b * Mp + M]

    ML = B * Mp - 3 * W2
    acc = _conv9_stacked(xpad_ref, w_ref, ML, W2)
    _store_stats(acc, B, ML, W, W2, Mp, M, y_ref, st_ref)


def _head_kernel(prev_ref, st_ref_in, g_ref, b_ref, p_ref, out_ref,
                 *, B, M, Mp, cnt):
    scale, shift = _bn_ss(st_ref_in, g_ref, b_ref, cnt)
    for b in range(B):
        act = jnp.maximum(prev_ref[pl.ds(b * Mp, M), :].astype(jnp.float32) * scale + shift, 0.0)
        out_ref[b] = jnp.dot(p_ref[...], act, preferred_element_type=jnp.float32)


def _conv_first(xfold, w27, B, H, W, od):
    S = xfold.shape[0]
    Cout = w27.shape[-1]
    W2 = W + 2
    Mp = (H + 3) * W2
    ML = B * Mp - 3 * W2
    return pl.pallas_call(
        partial(_conv1_kernel, B=B, H=H, W=W),
        out_shape=(jax.ShapeDtypeStruct((S, B * Mp, Cout), od),
                   jax.ShapeDtypeStruct((S, B, 2, Cout), jnp.float32)),
        grid=(S,),
        in_specs=[pl.BlockSpec((None, B * Mp, 9), lambda n: (n, 0, 0)),
                  pl.BlockSpec((27, Cout), lambda n: (0, 0))],
        out_specs=(pl.BlockSpec((None, B * Mp, Cout), lambda n: (n, 0, 0)),
                   pl.BlockSpec((None, B, 2, Cout), lambda n: (n, 0, 0, 0))),
        scratch_shapes=[pltpu.VMEM((ML, 27), jnp.bfloat16)],
        compiler_params=pltpu.CompilerParams(
            dimension_semantics=("parallel",), vmem_limit_bytes=VMEM_LIMIT),
    )(xfold, w27)


def _fused_pool_conv(y_prev, st_prev, gamma, beta, w9, B, Hp, Wp, cnt, od):
    Cp = y_prev.shape[-1]
    Cout = w9.shape[-1]
    W2p = Wp + 2
    Hc, Wc = (Hp - 3) // 2 + 1, (Wp - 3) // 2 + 1
    W2c = Wc + 2
    Mpc = (Hc + 3) * W2c
    N_TOT = (y_prev.shape[0] * y_prev.shape[1]) // ((Hp + 3) * W2p)
    S = N_TOT // B
    prev5 = y_prev.reshape(S, B, Hp + 3, W2p, Cp)
    st_in = st_prev.reshape(N_TOT, 2, Cp)
    y, st = pl.pallas_call(
        partial(_pool_conv_kernel, B=B, Hp=Hp, Wp=Wp, Hc=Hc, Wc=Wc, cnt=cnt),
        out_shape=(jax.ShapeDtypeStruct((S, B * Mpc, Cout), od),
                   jax.ShapeDtypeStruct((S, B, 2, Cout), jnp.float32)),
        grid=(S,),
        in_specs=[pl.BlockSpec((None, B, Hp + 3, W2p, Cp),
                               lambda n: (n, 0, 0, 0, 0)),
                  pl.BlockSpec((N_TOT, 2, Cp), lambda n: (0, 0, 0)),
                  pl.BlockSpec((1, Cp), lambda n: (0, 0)),
                  pl.BlockSpec((1, Cp), lambda n: (0, 0)),
                  pl.BlockSpec((9, Cp, Cout), lambda n: (0, 0, 0))],
        out_specs=(pl.BlockSpec((None, B * Mpc, Cout), lambda n: (n, 0, 0)),
                   pl.BlockSpec((None, B, 2, Cout), lambda n: (n, 0, 0, 0))),
        scratch_shapes=[pltpu.VMEM((Hp, W2p, Cp), jnp.float32),
                        pltpu.VMEM((Hp, Wc, Cp), jnp.float32),
                        pltpu.VMEM((B * Mpc, Cp), jnp.bfloat16)],
        compiler_params=pltpu.CompilerParams(
            dimension_semantics=("parallel",), vmem_limit_bytes=VMEM_LIMIT),
    )(prev5, st_in, gamma, beta, w9)
    return y, st, Hc, Wc


def _fused_conv(y_prev, st_prev, gamma, beta, w9, B, H, W, cnt, od):
    Cp = y_prev.shape[-1]
    Cout = w9.shape[-1]
    W2 = W + 2
    Mp = (H + 3) * W2
    N_TOT = (y_prev.shape[0] * y_prev.shape[1]) // Mp
    S = N_TOT // B
    yv = y_prev.reshape(S, B * Mp, Cp)
    st_in = st_prev.reshape(N_TOT, 2, Cp)
    return pl.pallas_call(
        partial(_conv_kernel, B=B, H=H, W=W, cnt=cnt),
        out_shape=(jax.ShapeDtypeStruct((S, B * Mp, Cout), od),
                   jax.ShapeDtypeStruct((S, B, 2, Cout), jnp.float32)),
        grid=(S,),
        in_specs=[pl.BlockSpec((None, B * Mp, Cp), lambda n: (n, 0, 0)),
                  pl.BlockSpec((N_TOT, 2, Cp), lambda n: (0, 0, 0)),
                  pl.BlockSpec((1, Cp), lambda n: (0, 0)),
                  pl.BlockSpec((1, Cp), lambda n: (0, 0)),
                  pl.BlockSpec((9, Cp, Cout), lambda n: (0, 0, 0))],
        out_specs=(pl.BlockSpec((None, B * Mp, Cout), lambda n: (n, 0, 0)),
                   pl.BlockSpec((None, B, 2, Cout), lambda n: (n, 0, 0, 0))),
        scratch_shapes=[pltpu.VMEM((B * Mp, Cp), jnp.bfloat16)],
        compiler_params=pltpu.CompilerParams(
            dimension_semantics=("parallel",), vmem_limit_bytes=VMEM_LIMIT),
    )(yv, st_in, gamma, beta, w9)


def _head_pool(y_prev, st_prev, gamma, beta, pmat, B, H, W, cnt):
    C = y_prev.shape[-1]
    P, M = pmat.shape
    Mp = (H + 3) * (W + 2)
    N_TOT = (y_prev.shape[0] * y_prev.shape[1]) // Mp
    S = N_TOT // B
    yv = y_prev.reshape(S, B * Mp, C)
    st_in = st_prev.reshape(N_TOT, 2, C)
    return pl.pallas_call(
        partial(_head_kernel, B=B, M=M, Mp=Mp, cnt=cnt),
        out_shape=jax.ShapeDtypeStruct((S, B, P, C), jnp.float32),
        grid=(S,),
        in_specs=[pl.BlockSpec((None, B * Mp, C), lambda n: (n, 0, 0)),
                  pl.BlockSpec((N_TOT, 2, C), lambda n: (0, 0, 0)),
                  pl.BlockSpec((1, C), lambda n: (0, 0)),
                  pl.BlockSpec((1, C), lambda n: (0, 0)),
                  pl.BlockSpec((P, M), lambda n: (0, 0))],
        out_specs=pl.BlockSpec((None, B, P, C), lambda n: (n, 0, 0, 0)),
        compiler_params=pltpu.CompilerParams(
            dimension_semantics=("parallel",), vmem_limit_bytes=VMEM_LIMIT),
    )(yv, st_in, gamma, beta, pmat)


def kernel(conv_w_0, conv_w_1, conv_w_2, conv_w_3, conv_w_4, conv_w_5,
           gamma_0, gamma_1, gamma_2, gamma_3, gamma_4, gamma_5,
           beta_0, beta_1, beta_2, beta_3, beta_4, beta_5,
           fc_w_perm, fc_b, pool_mat, x):
    conv_w = [conv_w_0, conv_w_1, conv_w_2, conv_w_3, conv_w_4, conv_w_5]
    gammas = [gamma_0, gamma_1, gamma_2, gamma_3, gamma_4, gamma_5]
    betas = [beta_0, beta_1, beta_2, beta_3, beta_4, beta_5]

    N, Cin, H, W = x.shape
    B1, B2, B3, B46, BH = 1, 1, 4, 16, 32
    S1 = N // B1
    W2 = W + 2
    Mp = (H + 3) * W2

    xh = jnp.transpose(x, (0, 2, 3, 1)).astype(jnp.float32)
    xp = jnp.pad(xh, ((0, 0), (1, 2), (1, 1), (0, 0))).astype(jnp.bfloat16)
    xp = xp.reshape(N, Mp, Cin)
    xpb = jnp.pad(xp, ((0, 0), (0, 2), (0, 0)))
    xfold = jnp.concatenate(
        [xpb[:, 0:Mp], xpb[:, 1:Mp + 1], xpb[:, 2:Mp + 2]], axis=2)
    xfold = xfold.reshape(S1, B1 * Mp, 3 * Cin)
    w27 = conv_w[0].reshape(9 * Cin, conv_w[0].shape[-1])

    ODS = [jnp.float32, jnp.float32, jnp.float32,
           jnp.float32, jnp.float32, jnp.float32]
    y, st = _conv_first(xfold, w27, B1, H, W, ODS[0])
    h, w = H, W

    for i, bi in ((1, B2), (2, B3)):
        y, st, h, w = _fused_pool_conv(y, st, gammas[i - 1], betas[i - 1],
                                       conv_w[i], bi, h, w, float(N * h * w), ODS[i])

    for i in (3, 4, 5):
        y, st = _fused_conv(y, st, gammas[i - 1], betas[i - 1],
                            conv_w[i], B46, h, w, float(N * h * w), ODS[i])

    pooled = _head_pool(y, st, gammas[5], betas[5], pool_mat, BH, h, w,
                        float(N * h * w))

    flat = pooled.reshape(N, -1)
    return flat @ fc_w_perm + fc_b[None, :]


# speedup vs baseline: 1.0114x; 1.0001x over previous
from functools import partial

import jax
import jax.numpy as jnp
from jax import lax
from jax.experimental import pallas as pl
from jax.experimental.pallas import tpu as pltpu

BN_EPS = 1e-5
VMEM_LIMIT = 64 * 1024 * 1024


def _bn_ss(st_ref, g_ref, b_ref, cnt):
    s1 = jnp.sum(st_ref[:, 0, :], axis=0, keepdims=True)
    s2 = jnp.sum(st_ref[:, 1, :], axis=0, keepdims=True)
    mean = s1 / cnt
    var = jnp.maximum(s2 / cnt - mean * mean, 0.0)
    scale = g_ref[...] * lax.rsqrt(var + BN_EPS)
    shift = b_ref[...] - mean * scale
    return scale, shift


def _conv9_stacked(xpad_ref, w_ref, ML, W2):
    Cout = w_ref.shape[-1]
    acc = jnp.zeros((ML, Cout), jnp.float32)
    for t in range(9):
        off = (t // 3) * W2 + (t % 3)
        acc = acc + jnp.dot(xpad_ref[pl.ds(off, ML), :], w_ref[t],
                            preferred_element_type=jnp.float32)
    return acc


def _store_stats(acc, B, ML, W, W2, Mp, M, y_ref, st_ref):
    y_ref[pl.ds(0, ML), :] = acc.astype(y_ref.dtype)
    r = lax.broadcasted_iota(jnp.int32, (M, 1), 0)
    cmask = (r % W2) < W
    for b in range(B):
        ymb = jnp.where(cmask, acc[b * Mp:b * Mp + M], 0.0)
        st_ref[b, 0:1, :] = jnp.sum(ymb, axis=0, keepdims=True)
        st_ref[b, 1:2, :] = jnp.sum(ymb * ymb, axis=0, keepdims=True)


def _conv1_kernel(x_ref, w_ref, y_ref, st_ref, xcol_ref, *, B, H, W):
    W2 = W + 2
    Mp = (H + 3) * W2
    M = H * W2
    ML = B * Mp - 3 * W2
    for dh in range(3):
        xcol_ref[:, 9 * dh:9 * (dh + 1)] = x_ref[pl.ds(dh * W2, ML), :]
    acc = jnp.dot(xcol_ref[...], w_ref[...], preferred_element_type=jnp.float32)
    _store_stats(acc, B, ML, W, W2, Mp, M, y_ref, st_ref)


def _pool_conv_kernel(prev_ref, st_ref_in, g_ref, b_ref, w_ref, y_ref, st_ref,
                      act_ref, rmax_ref, xpad_ref, *, B, Hp, Wp, Hc, Wc, cnt):
    scale, shift = _bn_ss(st_ref_in, g_ref, b_ref, cnt)
    W2c = Wc + 2
    Mpc = (Hc + 3) * W2c
    xpad_ref[...] = jnp.zeros_like(xpad_ref)

    for b in range(B):
        act_ref[...] = jnp.maximum(prev_ref[b, 0:Hp].astype(jnp.float32) * scale + shift, 0.0)
        cm = act_ref[:, pl.ds(0, Wc, stride=2), :]
        cm = jnp.maximum(cm, act_ref[:, pl.ds(1, Wc, stride=2), :])
        cm = jnp.maximum(cm, act_ref[:, pl.ds(2, Wc, stride=2), :])
        rmax_ref[...] = cm
        pooled = rmax_ref[pl.ds(0, Hc, stride=2), :, :]
        pooled = jnp.maximum(pooled, rmax_ref[pl.ds(1, Hc, stride=2), :, :])
        pooled = jnp.maximum(pooled, rmax_ref[pl.ds(2, Hc, stride=2), :, :])
        pooled = pooled.astype(jnp.bfloat16)
        base = b * Mpc
        for h in range(Hc):
            xpad_ref[pl.ds(base + (h + 1) * W2c + 1, Wc), :] = pooled[h]

    ML = B * Mpc - 3 * W2c
    acc = _conv9_stacked(xpad_ref, w_ref, ML, W2c)
    _store_stats(acc, B, ML, Wc, W2c, Mpc, Hc * W2c, y_ref, st_ref)


def _conv_kernel(prev_ref, st_ref_in, g_ref, b_ref, w_ref, y_ref, st_ref,
                 xpad_ref, *, B, H, W, cnt):
    W2 = W + 2
    M = H * W2
    Mp = (H + 3) * W2
    scale, shift = _bn_ss(st_ref_in, g_ref, b_ref, cnt)
    ML0 = B * Mp
    r = lax.broadcasted_iota(jnp.int32, (ML0, 1), 0)
    norm = jnp.where((r % W2) < W,
                     jnp.maximum(prev_ref[...].astype(jnp.float32) * scale + shift, 0.0),
                     0.0).astype(jnp.bfloat16)

    zb = jnp.zeros((W2 + 1, norm.shape[-1]), jnp.bfloat16)
    zt = jnp.zeros((2 * W2 - 1, norm.shape[-1]), jnp.bfloat16)
    for b in range(B):
        xpad_ref[pl.ds(b * Mp, W2 + 1), :] = zb
        xpad_ref[pl.ds(b * Mp + W2 + 1 + M, 2 * W2 - 1), :] = zt
        xpad_ref[pl.ds(b * Mp + W2 + 1, M), :] = norm[b * Mp:b * Mp + M]

    ML = B * Mp - 3 * W2
    acc = _conv9_stacked(xpad_ref, w_ref, ML, W2)
    _store_stats(acc, B, ML, W, W2, Mp, M, y_ref, st_ref)


def _head_kernel(prev_ref, st_ref_in, g_ref, b_ref, p_ref, out_ref,
                 *, B, M, Mp, cnt):
    scale, shift = _bn_ss(st_ref_in, g_ref, b_ref, cnt)
    for b in range(B):
        act = jnp.maximum(prev_ref[pl.ds(b * Mp, M), :].astype(jnp.float32) * scale + shift, 0.0)
        out_ref[b] = jnp.dot(p_ref[...], act, preferred_element_type=jnp.float32)


def _conv_first(xfold, w27, B, H, W, od):
    S = xfold.shape[0]
    Cout = w27.shape[-1]
    W2 = W + 2
    Mp = (H + 3) * W2
    ML = B * Mp - 3 * W2
    return pl.pallas_call(
        partial(_conv1_kernel, B=B, H=H, W=W),
        out_shape=(jax.ShapeDtypeStruct((S, B * Mp, Cout), od),
                   jax.ShapeDtypeStruct((S, B, 2, Cout), jnp.float32)),
        grid=(S,),
        in_specs=[pl.BlockSpec((None, B * Mp, 9), lambda n: (n, 0, 0)),
                  pl.BlockSpec((27, Cout), lambda n: (0, 0))],
        out_specs=(pl.BlockSpec((None, B * Mp, Cout), lambda n: (n, 0, 0)),
                   pl.BlockSpec((None, B, 2, Cout), lambda n: (n, 0, 0, 0))),
        scratch_shapes=[pltpu.VMEM((ML, 27), jnp.bfloat16)],
        compiler_params=pltpu.CompilerParams(
            dimension_semantics=("parallel",), vmem_limit_bytes=VMEM_LIMIT),
    )(xfold, w27)


def _fused_pool_conv(y_prev, st_prev, gamma, beta, w9, B, Hp, Wp, cnt, od):
    Cp = y_prev.shape[-1]
    Cout = w9.shape[-1]
    W2p = Wp + 2
    Hc, Wc = (Hp - 3) // 2 + 1, (Wp - 3) // 2 + 1
    W2c = Wc + 2
    Mpc = (Hc + 3) * W2c
    N_TOT = (y_prev.shape[0] * y_prev.shape[1]) // ((Hp + 3) * W2p)
    S = N_TOT // B
    prev5 = y_prev.reshape(S, B, Hp + 3, W2p, Cp)
    st_in = st_prev.reshape(N_TOT, 2, Cp)
    y, st = pl.pallas_call(
        partial(_pool_conv_kernel, B=B, Hp=Hp, Wp=Wp, Hc=Hc, Wc=Wc, cnt=cnt),
        out_shape=(jax.ShapeDtypeStruct((S, B * Mpc, Cout), od),
                   jax.ShapeDtypeStruct((S, B, 2, Cout), jnp.float32)),
        grid=(S,),
        in_specs=[pl.BlockSpec((None, B, Hp + 3, W2p, Cp),
                               lambda n: (n, 0, 0, 0, 0)),
                  pl.BlockSpec((N_TOT, 2, Cp), lambda n: (0, 0, 0)),
                  pl.BlockSpec((1, Cp), lambda n: (0, 0)),
                  pl.BlockSpec((1, Cp), lambda n: (0, 0)),
                  pl.BlockSpec((9, Cp, Cout), lambda n: (0, 0, 0))],
        out_specs=(pl.BlockSpec((None, B * Mpc, Cout), lambda n: (n, 0, 0)),
                   pl.BlockSpec((None, B, 2, Cout), lambda n: (n, 0, 0, 0))),
        scratch_shapes=[pltpu.VMEM((Hp, W2p, Cp), jnp.float32),
                        pltpu.VMEM((Hp, Wc, Cp), jnp.float32),
                        pltpu.VMEM((B * Mpc, Cp), jnp.bfloat16)],
        compiler_params=pltpu.CompilerParams(
            dimension_semantics=("parallel",), vmem_limit_bytes=VMEM_LIMIT),
    )(prev5, st_in, gamma, beta, w9)
    return y, st, Hc, Wc


def _fused_conv(y_prev, st_prev, gamma, beta, w9, B, H, W, cnt, od):
    Cp = y_prev.shape[-1]
    Cout = w9.shape[-1]
    W2 = W + 2
    Mp = (H + 3) * W2
    N_TOT = (y_prev.shape[0] * y_prev.shape[1]) // Mp
    S = N_TOT // B
    yv = y_prev.reshape(S, B * Mp, Cp)
    st_in = st_prev.reshape(N_TOT, 2, Cp)
    return pl.pallas_call(
        partial(_conv_kernel, B=B, H=H, W=W, cnt=cnt),
        out_shape=(jax.ShapeDtypeStruct((S, B * Mp, Cout), od),
                   jax.ShapeDtypeStruct((S, B, 2, Cout), jnp.float32)),
        grid=(S,),
        in_specs=[pl.BlockSpec((None, B * Mp, Cp), lambda n: (n, 0, 0)),
                  pl.BlockSpec((N_TOT, 2, Cp), lambda n: (0, 0, 0)),
                  pl.BlockSpec((1, Cp), lambda n: (0, 0)),
                  pl.BlockSpec((1, Cp), lambda n: (0, 0)),
                  pl.BlockSpec((9, Cp, Cout), lambda n: (0, 0, 0))],
        out_specs=(pl.BlockSpec((None, B * Mp, Cout), lambda n: (n, 0, 0)),
                   pl.BlockSpec((None, B, 2, Cout), lambda n: (n, 0, 0, 0))),
        scratch_shapes=[pltpu.VMEM((B * Mp, Cp), jnp.bfloat16)],
        compiler_params=pltpu.CompilerParams(
            dimension_semantics=("parallel",), vmem_limit_bytes=VMEM_LIMIT),
    )(yv, st_in, gamma, beta, w9)


def _head_pool(y_prev, st_prev, gamma, beta, pmat, B, H, W, cnt):
    C = y_prev.shape[-1]
    P, M = pmat.shape
    Mp = (H + 3) * (W + 2)
    N_TOT = (y_prev.shape[0] * y_prev.shape[1]) // Mp
    S = N_TOT // B
    yv = y_prev.reshape(S, B * Mp, C)
    st_in = st_prev.reshape(N_TOT, 2, C)
    return pl.pallas_call(
        partial(_head_kernel, B=B, M=M, Mp=Mp, cnt=cnt),
        out_shape=jax.ShapeDtypeStruct((S, B, P, C), jnp.float32),
        grid=(S,),
        in_specs=[pl.BlockSpec((None, B * Mp, C), lambda n: (n, 0, 0)),
                  pl.BlockSpec((N_TOT, 2, C), lambda n: (0, 0, 0)),
                  pl.BlockSpec((1, C), lambda n: (0, 0)),
                  pl.BlockSpec((1, C), lambda n: (0, 0)),
                  pl.BlockSpec((P, M), lambda n: (0, 0))],
        out_specs=pl.BlockSpec((None, B, P, C), lambda n: (n, 0, 0, 0)),
        compiler_params=pltpu.CompilerParams(
            dimension_semantics=("parallel",), vmem_limit_bytes=VMEM_LIMIT),
    )(yv, st_in, gamma, beta, pmat)


def kernel(conv_w_0, conv_w_1, conv_w_2, conv_w_3, conv_w_4, conv_w_5,
           gamma_0, gamma_1, gamma_2, gamma_3, gamma_4, gamma_5,
           beta_0, beta_1, beta_2, beta_3, beta_4, beta_5,
           fc_w_perm, fc_b, pool_mat, x):
    conv_w = [conv_w_0, conv_w_1, conv_w_2, conv_w_3, conv_w_4, conv_w_5]
    gammas = [gamma_0, gamma_1, gamma_2, gamma_3, gamma_4, gamma_5]
    betas = [beta_0, beta_1, beta_2, beta_3, beta_4, beta_5]

    N, Cin, H, W = x.shape
    B1, B2, B3, B46, BH = 1, 1, min(4, N), min(16, N), min(32, N)
    S1 = N // B1
    W2 = W + 2
    Mp = (H + 3) * W2

    xh = jnp.transpose(x, (0, 2, 3, 1)).astype(jnp.float32)
    xp = jnp.pad(xh, ((0, 0), (1, 2), (1, 1), (0, 0))).astype(jnp.bfloat16)
    xp = xp.reshape(N, Mp, Cin)
    xpb = jnp.pad(xp, ((0, 0), (0, 2), (0, 0)))
    xfold = jnp.concatenate(
        [xpb[:, 0:Mp], xpb[:, 1:Mp + 1], xpb[:, 2:Mp + 2]], axis=2)
    xfold = xfold.reshape(S1, B1 * Mp, 3 * Cin)
    w27 = conv_w[0].reshape(9 * Cin, conv_w[0].shape[-1])

    ODS = [jnp.float32, jnp.float32, jnp.float32,
           jnp.float32, jnp.float32, jnp.float32]
    y, st = _conv_first(xfold, w27, B1, H, W, ODS[0])
    h, w = H, W

    for i, bi in ((1, B2), (2, B3)):
        y, st, h, w = _fused_pool_conv(y, st, gammas[i - 1], betas[i - 1],
                                       conv_w[i], bi, h, w, float(N * h * w), ODS[i])

    for i in (3, 4, 5):
        y, st = _fused_conv(y, st, gammas[i - 1], betas[i - 1],
                            conv_w[i], B46, h, w, float(N * h * w), ODS[i])

    pooled = _head_pool(y, st, gammas[5], betas[5], pool_mat, BH, h, w,
                        float(N * h * w))

    flat = pooled.reshape(N, -1)
    return flat @ fc_w_perm + fc_b[None, :]
